# Initial kernel scaffold; baseline (speedup 1.0000x reference)
import jax
import jax.numpy as jnp
from jax.experimental import pallas as pl


def kernel(X, W_switch, b_switch, Wq, bq, Wk, bk, Wv, bv, Wff, bff):
    raise NotImplementedError("write your pallas kernel here")


# speedup vs baseline: 3.2097x; 3.2097x over previous
import math

import jax
import jax.numpy as jnp
from jax.experimental import pallas as pl
from jax.experimental.pallas import tpu as pltpu

B, S, D = 64, 197, 768
E = 12
HD = 128
TOPK = 4
INV_SCALE = 1.0 / math.sqrt(D // E)

ROUTER_CHUNKS = 6
CHUNK = (S * D) // ROUTER_CHUNKS


def _router_body(x_ref, w_ref, b_ref, ids_ref, acc_ref):
    i = pl.program_id(0)

    @pl.when(i == 0)
    def _init():
        acc_ref[...] = jnp.zeros_like(acc_ref)

    acc_ref[...] += jax.lax.dot_general(
        x_ref[...], w_ref[...], (((1,), (0,)), ((), ())),
        preferred_element_type=jnp.float32,
        precision=jax.lax.Precision.HIGHEST)

    @pl.when(i == ROUTER_CHUNKS - 1)
    def _topk():
        logits = acc_ref[...] + b_ref[...]
        lane = jax.lax.broadcasted_iota(jnp.int32, (B, E), 1)
        vals = logits
        cols = []
        for _ in range(TOPK):
            m = jnp.max(vals, axis=1, keepdims=True)
            idx = jnp.min(jnp.where(vals == m, lane, E + 1), axis=1,
                          keepdims=True)
            cols.append(idx)
            vals = jnp.where(lane == idx, -1e30, vals)
        ids_ref[...] = jnp.concatenate(cols, axis=1)


def _moe_body(ids_ref, x_ref, wq_ref, wk_ref, wv_ref, wff_ref,
              bq_ref, bk_ref, bv_ref, bff_ref, out_ref):
    i = pl.program_id(0)
    x = x_ref[0].astype(jnp.bfloat16)
    for t in range(TOPK):
        e = ids_ref[i, t]
        q = (jax.lax.dot_general(
            x, wq_ref[e], (((1,), (0,)), ((), ())),
            preferred_element_type=jnp.float32) + bq_ref[e]
             ).astype(jnp.bfloat16)
        k = (jax.lax.dot_general(
            x, wk_ref[e], (((1,), (0,)), ((), ())),
            preferred_element_type=jnp.float32) + bk_ref[e]
             ).astype(jnp.bfloat16)
        v = (jax.lax.dot_general(
            x, wv_ref[e], (((1,), (0,)), ((), ())),
            preferred_element_type=jnp.float32) + bv_ref[e]
             ).astype(jnp.bfloat16)
        s = jax.lax.dot_general(
            q, k, (((1,), (1,)), ((), ())),
            preferred_element_type=jnp.float32) * INV_SCALE
        m = jnp.max(s, axis=1, keepdims=True)
        p = jnp.exp(s - m)
        p = (p / jnp.sum(p, axis=1, keepdims=True)).astype(jnp.bfloat16)
        ctx = jax.lax.dot_general(
            p, v, (((1,), (0,)), ((), ())),
            preferred_element_type=jnp.float32).astype(jnp.bfloat16)
        o = jax.lax.dot_general(
            ctx, wff_ref[e], (((1,), (0,)), ((), ())),
            preferred_element_type=jnp.float32)
        if t == 0:
            out_ref[0] = o + TOPK * bff_ref[0]
        else:
            out_ref[0] += o


def kernel(X, W_switch, b_switch, Wq, bq, Wk, bk, Wv, bv, Wff, bff):
    Xf = X.reshape(B, S * D)
    ids = pl.pallas_call(
        _router_body,
        grid=(ROUTER_CHUNKS,),
        in_specs=[
            pl.BlockSpec((B, CHUNK), lambda i: (0, i)),
            pl.BlockSpec((CHUNK, E), lambda i: (i, 0)),
            pl.BlockSpec((1, E), lambda i: (0, 0)),
        ],
        out_specs=pl.BlockSpec((B, TOPK), lambda i: (0, 0)),
        out_shape=jax.ShapeDtypeStruct((B, TOPK), jnp.int32),
        scratch_shapes=[pltpu.VMEM((B, E), jnp.float32)],
    )(Xf, W_switch, b_switch.reshape(1, E))

    grid_spec = pltpu.PrefetchScalarGridSpec(
        num_scalar_prefetch=1,
        grid=(B,),
        in_specs=[
            pl.BlockSpec((1, S, D), lambda i, ids: (i, 0, 0)),
            pl.BlockSpec((E, D, HD), lambda i, ids: (0, 0, 0)),
            pl.BlockSpec((E, D, HD), lambda i, ids: (0, 0, 0)),
            pl.BlockSpec((E, D, HD), lambda i, ids: (0, 0, 0)),
            pl.BlockSpec((E, HD, D), lambda i, ids: (0, 0, 0)),
            pl.BlockSpec((E, 1, HD), lambda i, ids: (0, 0, 0)),
            pl.BlockSpec((E, 1, HD), lambda i, ids: (0, 0, 0)),
            pl.BlockSpec((E, 1, HD), lambda i, ids: (0, 0, 0)),
            pl.BlockSpec((1, D), lambda i, ids: (0, 0)),
        ],
        out_specs=pl.BlockSpec((1, S, D), lambda i, ids: (i, 0, 0)),
    )
    out = pl.pallas_call(
        _moe_body,
        grid_spec=grid_spec,
        out_shape=jax.ShapeDtypeStruct((B, S, D), jnp.float32),
    )(ids, X,
      Wq.astype(jnp.bfloat16), Wk.astype(jnp.bfloat16),
      Wv.astype(jnp.bfloat16), Wff.astype(jnp.bfloat16),
      bq.reshape(E, 1, HD), bk.reshape(E, 1, HD), bv.reshape(E, 1, HD),
      bff.reshape(1, D))
    return out


# speedup vs baseline: 3.9677x; 1.2362x over previous
import math

import jax
import jax.numpy as jnp
from jax.experimental import pallas as pl
from jax.experimental.pallas import tpu as pltpu

B, S, D = 64, 197, 768
E = 12
HD = 128
TOPK = 4
INV_SCALE = 1.0 / math.sqrt(D // E)

ROUTER_CHUNKS = 6
CHUNK = (S * D) // ROUTER_CHUNKS


def _router_body(x_ref, w_ref, b_ref, ids_ref, acc_ref):
    i = pl.program_id(0)

    @pl.when(i == 0)
    def _init():
        acc_ref[...] = jnp.zeros_like(acc_ref)

    acc_ref[...] += jax.lax.dot_general(
        x_ref[...], w_ref[...], (((1,), (0,)), ((), ())),
        preferred_element_type=jnp.float32,
        precision=jax.lax.Precision.HIGHEST)

    @pl.when(i == ROUTER_CHUNKS - 1)
    def _topk():
        logits = acc_ref[...] + b_ref[...]
        lane = jax.lax.broadcasted_iota(jnp.int32, (B, E), 1)
        vals = logits
        cols = []
        for _ in range(TOPK):
            m = jnp.max(vals, axis=1, keepdims=True)
            idx = jnp.min(jnp.where(vals == m, lane, E + 1), axis=1,
                          keepdims=True)
            cols.append(idx)
            vals = jnp.where(lane == idx, -1e30, vals)
        ids_ref[...] = jnp.concatenate(cols, axis=1)


def _prep_body(wq_ref, wk_ref, wv_ref, wff_ref, bq_ref, bk_ref, bv_ref,
               wqkv_ref, bqkv_ref, wffo_ref):
    wqkv_ref[0, :, 0:HD] = wq_ref[0].astype(jnp.bfloat16)
    wqkv_ref[0, :, HD:2 * HD] = wk_ref[0].astype(jnp.bfloat16)
    wqkv_ref[0, :, 2 * HD:3 * HD] = wv_ref[0].astype(jnp.bfloat16)
    wffo_ref[0] = wff_ref[0].astype(jnp.bfloat16)
    bqkv_ref[0, :, 0:HD] = bq_ref[0]
    bqkv_ref[0, :, HD:2 * HD] = bk_ref[0]
    bqkv_ref[0, :, 2 * HD:3 * HD] = bv_ref[0]


def _moe_body(ids_ref, x_ref, wqkv_ref, wff_ref, bqkv_ref, bff_ref,
              out_ref):
    i = pl.program_id(0)
    x = x_ref[0].astype(jnp.bfloat16)
    ctxs = []
    wffs = []
    for t in range(TOPK):
        e = ids_ref[i, t]
        qkv = jax.lax.dot_general(
            x, wqkv_ref[e], (((1,), (0,)), ((), ())),
            preferred_element_type=jnp.float32) + bqkv_ref[e]
        q = qkv[:, 0:HD].astype(jnp.bfloat16)
        k = qkv[:, HD:2 * HD].astype(jnp.bfloat16)
        v = qkv[:, 2 * HD:3 * HD].astype(jnp.bfloat16)
        s = jax.lax.dot_general(
            q, k, (((1,), (1,)), ((), ())),
            preferred_element_type=jnp.float32) * INV_SCALE
        m = jnp.max(s, axis=1, keepdims=True)
        p = jnp.exp(s - m)
        p = (p / jnp.sum(p, axis=1, keepdims=True)).astype(jnp.bfloat16)
        ctxs.append(jax.lax.dot_general(
            p, v, (((1,), (0,)), ((), ())),
            preferred_element_type=jnp.float32).astype(jnp.bfloat16))
        wffs.append(wff_ref[e])
    ctx_all = jnp.concatenate(ctxs, axis=1)
    wff_all = jnp.concatenate(wffs, axis=0)
    out_ref[0] = jax.lax.dot_general(
        ctx_all, wff_all, (((1,), (0,)), ((), ())),
        preferred_element_type=jnp.float32) + TOPK * bff_ref[0]


def kernel(X, W_switch, b_switch, Wq, bq, Wk, bk, Wv, bv, Wff, bff):
    Xf = X.reshape(B, S * D)
    ids = pl.pallas_call(
        _router_body,
        grid=(ROUTER_CHUNKS,),
        in_specs=[
            pl.BlockSpec((B, CHUNK), lambda i: (0, i)),
            pl.BlockSpec((CHUNK, E), lambda i: (i, 0)),
            pl.BlockSpec((1, E), lambda i: (0, 0)),
        ],
        out_specs=pl.BlockSpec((B, TOPK), lambda i: (0, 0)),
        out_shape=jax.ShapeDtypeStruct((B, TOPK), jnp.int32),
        scratch_shapes=[pltpu.VMEM((B, E), jnp.float32)],
    )(Xf, W_switch, b_switch.reshape(1, E))

    wqkv, bqkv, wffb = pl.pallas_call(
        _prep_body,
        grid=(E,),
        in_specs=[
            pl.BlockSpec((1, D, HD), lambda i: (i, 0, 0)),
            pl.BlockSpec((1, D, HD), lambda i: (i, 0, 0)),
            pl.BlockSpec((1, D, HD), lambda i: (i, 0, 0)),
            pl.BlockSpec((1, HD, D), lambda i: (i, 0, 0)),
            pl.BlockSpec((1, 1, HD), lambda i: (i, 0, 0)),
            pl.BlockSpec((1, 1, HD), lambda i: (i, 0, 0)),
            pl.BlockSpec((1, 1, HD), lambda i: (i, 0, 0)),
        ],
        out_specs=[
            pl.BlockSpec((1, D, 3 * HD), lambda i: (i, 0, 0)),
            pl.BlockSpec((1, 1, 3 * HD), lambda i: (i, 0, 0)),
            pl.BlockSpec((1, HD, D), lambda i: (i, 0, 0)),
        ],
        out_shape=[
            jax.ShapeDtypeStruct((E, D, 3 * HD), jnp.bfloat16),
            jax.ShapeDtypeStruct((E, 1, 3 * HD), jnp.float32),
            jax.ShapeDtypeStruct((E, HD, D), jnp.bfloat16),
        ],
    )(Wq, Wk, Wv, Wff,
      bq.reshape(E, 1, HD), bk.reshape(E, 1, HD), bv.reshape(E, 1, HD))

    grid_spec = pltpu.PrefetchScalarGridSpec(
        num_scalar_prefetch=1,
        grid=(B,),
        in_specs=[
            pl.BlockSpec((1, S, D), lambda i, ids: (i, 0, 0)),
            pl.BlockSpec((E, D, 3 * HD), lambda i, ids: (0, 0, 0)),
            pl.BlockSpec((E, HD, D), lambda i, ids: (0, 0, 0)),
            pl.BlockSpec((E, 1, 3 * HD), lambda i, ids: (0, 0, 0)),
            pl.BlockSpec((1, D), lambda i, ids: (0, 0)),
        ],
        out_specs=pl.BlockSpec((1, S, D), lambda i, ids: (i, 0, 0)),
    )
    out = pl.pallas_call(
        _moe_body,
        grid_spec=grid_spec,
        out_shape=jax.ShapeDtypeStruct((B, S, D), jnp.float32),
    )(ids, X, wqkv, wffb, bqkv, bff.reshape(1, D))
    return out
